# baseline (speedup 1.0000x reference)
import functools

import jax
import jax.numpy as jnp
from jax import lax
from jax.experimental import pallas as pl
from jax.experimental.pallas import tpu as pltpu
from jax.experimental.pallas import tpu_sc as plsc

NC = 2
NS = 16
NW = NC * NS
L = 16

B = 16384
E = 64
TROWS = 500000
BPW = B // NW
NCHUNK = 4
CHUNK = BPW // NCHUNK
NBLK = CHUNK // L


def _sc_body(uidx_hbm, cidx_hbm, utab_hbm, ctab_hbm, w_hbm, b_hbm, out_hbm,
             uidx_v, cidx_v, uphys_v, cphys_v, ubuf, cbuf, w_v, b_v, out_v,
             sems):
    c = lax.axis_index("c")
    s = lax.axis_index("s")
    wid = s * NC + c
    base = wid * BPW

    pltpu.sync_copy(uidx_hbm.at[pl.ds(base, BPW)], uidx_v)
    pltpu.sync_copy(cidx_hbm.at[pl.ds(base, BPW)], cidx_v)
    pltpu.sync_copy(w_hbm, w_v)
    pltpu.sync_copy(b_hbm, b_v)

    def phys_body(k, carry):
        sl = pl.ds(k * L, L)
        uphys_v[sl] = lax.shift_right_logical(uidx_v[sl], 1)
        cphys_v[sl] = lax.shift_right_logical(cidx_v[sl], 1)
        return carry
    lax.fori_loop(0, BPW // L, phys_body, 0)

    def fire(j):
        buf = j % 2
        cu = pltpu.async_copy(
            utab_hbm.at[uphys_v.at[pl.ds(j * CHUNK, CHUNK)]], ubuf.at[buf],
            sems.at[buf, 0])
        cc = pltpu.async_copy(
            ctab_hbm.at[cphys_v.at[pl.ds(j * CHUNK, CHUNK)]], cbuf.at[buf],
            sems.at[buf, 1])
        return cu, cc

    wv = w_v[...]
    bv = b_v[...]
    iota = lax.iota(jnp.int32, L)

    def compute(j):
        buf = j % 2
        ub = ubuf.at[buf]
        cb = cbuf.at[buf]

        def blk_body(k, carry):
            row = k * L + iota
            g = pl.ds(j * CHUNK + k * L, L)
            ucol = (uidx_v[g] & 1) << 6
            ccol = (cidx_v[g] & 1) << 6
            acc = jnp.zeros((L,), jnp.float32)
            for d in range(E):
                gu = plsc.load_gather(ub, [row, ucol + d])
                gc = plsc.load_gather(cb, [row, ccol + d])
                acc = acc + gu * gc
            x = acc * wv + bv
            out_v[g] = 1.0 / (1.0 + jnp.exp(-x))
            return carry

        lax.fori_loop(0, NBLK, blk_body, 0)

    cps = {0: fire(0)}
    for j in range(NCHUNK):
        if j + 1 < NCHUNK:
            cps[j + 1] = fire(j + 1)
        cps[j][0].wait()
        cps[j][1].wait()
        compute(j)

    pltpu.sync_copy(out_v, out_hbm.at[pl.ds(base, BPW)])


@jax.jit
def _run(uidx, cidx, user_table, content_table, wvec, bvec):
    mesh = plsc.VectorSubcoreMesh(
        core_axis_name="c", subcore_axis_name="s",
        num_cores=NC, num_subcores=NS)
    return pl.kernel(
        _sc_body,
        out_type=jax.ShapeDtypeStruct((B,), jnp.float32),
        mesh=mesh,
        compiler_params=pltpu.CompilerParams(
            needs_layout_passes=False, use_tc_tiling_on_sc=True),
        scratch_types=[
            pltpu.VMEM((BPW,), jnp.int32),
            pltpu.VMEM((BPW,), jnp.int32),
            pltpu.VMEM((BPW,), jnp.int32),
            pltpu.VMEM((BPW,), jnp.int32),
            pltpu.VMEM((2, CHUNK, 2 * E), jnp.float32),
            pltpu.VMEM((2, CHUNK, 2 * E), jnp.float32),
            pltpu.VMEM((L,), jnp.float32),
            pltpu.VMEM((L,), jnp.float32),
            pltpu.VMEM((BPW,), jnp.float32),
            pltpu.SemaphoreType.DMA((2, 2)),
        ],
    )(uidx, cidx, user_table, content_table, wvec, bvec)


def kernel(inputs, user_table, content_table, dense_w, dense_b):
    uidx = inputs[:, 0]
    cidx = inputs[:, 1]
    utab = user_table.reshape(TROWS, 2 * E)
    ctab = content_table.reshape(TROWS, 2 * E)
    wvec = jnp.full((L,), dense_w[0, 0], jnp.float32)
    bvec = jnp.full((L,), dense_b[0], jnp.float32)
    out = _run(uidx, cidx, utab, ctab, wvec, bvec)
    return out.reshape(B, 1)
